# Initial kernel scaffold; baseline (speedup 1.0000x reference)
import jax
import jax.numpy as jnp
from jax.experimental import pallas as pl


def kernel(x, W):
    raise NotImplementedError("write your pallas kernel here")


# speedup vs baseline: 2.2010x; 2.2010x over previous
import jax
import jax.numpy as jnp
from jax.experimental import pallas as pl
from jax.experimental.pallas import tpu as pltpu

B, S, D, E, K = 4, 8192, 768, 8, 2
ALPHA = 0.01
EPS = 1e-20
T = 2048
N_TOK = B * S
GRID = N_TOK // T
BLOCKS_PER_BATCH = S // T
NEG_INF = float("-inf")


def _router_kernel(x_ref, w_ref, idx_ref, wgt_ref, aux_ref, acc_c, acc_s):
    step = pl.program_id(0)
    xb = x_ref[...]
    w = w_ref[...]
    lt = jax.lax.dot_general(
        xb, w, (((1,), (1,)), ((), ())),
        preferred_element_type=jnp.float32)

    m1 = jnp.max(lt, axis=-1, keepdims=True)
    lane = jax.lax.broadcasted_iota(jnp.int32, (T, E), 1)
    i1 = jnp.argmax(lt, axis=-1).astype(jnp.int32)
    masked = jnp.where(lane == i1[:, None], NEG_INF, lt)
    m2 = jnp.max(masked, axis=-1, keepdims=True)
    i2 = jnp.argmax(masked, axis=-1).astype(jnp.int32)

    p = jnp.exp(lt - m1)
    sump = jnp.sum(p, axis=-1, keepdims=True)
    p2 = jnp.exp(m2 - m1)
    denom = 1.0 + p2[:, 0] + EPS
    w1 = 1.0 / denom
    w2 = p2[:, 0] / denom

    idx_ref[...] = jnp.stack([i1, i2], axis=1)
    wgt_ref[...] = jnp.stack([w1, w2], axis=1)

    scores = p / sump
    s_blk = jnp.sum(scores, axis=0)
    onehot = (lane == i1[:, None]).astype(jnp.float32) + (
        lane == i2[:, None]).astype(jnp.float32)
    c_blk = jnp.sum(onehot, axis=0)

    b_id = step // BLOCKS_PER_BATCH

    @pl.when(step == 0)
    def _():
        acc_c[...] = jnp.zeros_like(acc_c)
        acc_s[...] = jnp.zeros_like(acc_s)

    acc_c[b_id, :] += c_blk
    acc_s[b_id, :] += s_blk

    @pl.when(step == GRID - 1)
    def _():
        ce = acc_c[...] / (S * K / E)
        sm = acc_s[...] / S
        aux_ref[0, 0] = jnp.sum(ce * sm) / B * ALPHA


def kernel(x, W):
    xf = x.reshape(N_TOK, D)
    idx, wgt, aux = pl.pallas_call(
        _router_kernel,
        grid=(GRID,),
        in_specs=[
            pl.BlockSpec((T, D), lambda i: (i, 0)),
            pl.BlockSpec((E, D), lambda i: (0, 0)),
        ],
        out_specs=[
            pl.BlockSpec((T, K), lambda i: (i, 0)),
            pl.BlockSpec((T, K), lambda i: (i, 0)),
            pl.BlockSpec(memory_space=pltpu.SMEM),
        ],
        out_shape=[
            jax.ShapeDtypeStruct((N_TOK, K), jnp.int32),
            jax.ShapeDtypeStruct((N_TOK, K), jnp.float32),
            jax.ShapeDtypeStruct((1, 1), jnp.float32),
        ],
        scratch_shapes=[
            pltpu.VMEM((B, E), jnp.float32),
            pltpu.VMEM((B, E), jnp.float32),
        ],
    )(xf, W)
    return idx, wgt, aux.reshape(())
